# Initial kernel scaffold; baseline (speedup 1.0000x reference)
import jax
import jax.numpy as jnp
from jax.experimental import pallas as pl


def kernel(x, edge_index, fm_w, fm_b, fm_v, Wl1, bl1, Wr1, g1, be1, Wl2, bl2, Wr2, g2, be2, Wl3, bl3, Wr3, w1, b1, w2, b2):
    raise NotImplementedError("write your pallas kernel here")


# speedup vs baseline: 7.1809x; 7.1809x over previous
import functools

import jax
import jax.numpy as jnp
from jax import lax
from jax.experimental import pallas as pl
from jax.experimental.pallas import tpu as pltpu
from jax.experimental.pallas import tpu_sc as plsc

_N = 50000
_E = 1600000
_T = 4
_H = 64

_NC = 2
_NS = 16
_NP = 50048
_RPS = _NP // _NS
_K = 80

_mesh = plsc.VectorSubcoreMesh(core_axis_name="c", subcore_axis_name="s")


def _seg_kernel(width, feature_split, has_gather):
    if feature_split:
        epw = _E // _NS
    else:
        epw = _E // (_NC * _NS)
    n_chunks = epw // _K
    assert n_chunks * _K == epw

    scratch = [
        pltpu.VMEM((_K,), jnp.int32),
        pltpu.VMEM((1, _K), jnp.int32),
        pltpu.VMEM((_K, width), jnp.float32),
        pltpu.VMEM_SHARED((_NP, width), jnp.float32),
        pltpu.SemaphoreType.DMA,
    ]

    @functools.partial(
        pl.kernel,
        out_type=jax.ShapeDtypeStruct((_NC, _NP, width), jnp.float32),
        mesh=_mesh,
        scratch_types=scratch,
        compiler_params=pltpu.CompilerParams(use_tc_tiling_on_sc=False),
        name=f"sc_seg_w{width}_{'fs' if feature_split else 'es'}",
    )
    def kern(tbl, gidx_hbm, sidx_hbm, zrows, out, gv, sv, rows, acc, sem):
        c = lax.axis_index("c")
        s = lax.axis_index("s")
        pltpu.sync_copy(zrows, acc.at[pl.ds(s * _RPS, _RPS)])
        if not has_gather:
            pltpu.sync_copy(tbl, rows)
        plsc.subcore_barrier()

        if feature_split:
            base0 = s * epw
        else:
            base0 = (s * _NC + c) * epw

        def chunk(i, _):
            base = base0 + i * _K
            pltpu.sync_copy(sidx_hbm.at[pl.ds(base, _K)], sv.at[0])
            if has_gather:
                pltpu.sync_copy(gidx_hbm.at[pl.ds(base, _K)], gv)
                if feature_split:
                    pltpu.async_copy(tbl.at[c].at[gv], rows, sem).wait()
                else:
                    pltpu.async_copy(tbl.at[gv], rows, sem).wait()
            pltpu.sync_copy(rows, acc.at[sv.at[0]], add=True)
            return _

        lax.fori_loop(0, n_chunks, chunk, 0)
        plsc.subcore_barrier()
        sl = pl.ds(s * _RPS, _RPS)
        pltpu.sync_copy(acc.at[sl], out.at[c].at[sl])

    return kern


_deg_kernel = _seg_kernel(8, feature_split=False, has_gather=False)
_cvec_kernel = _seg_kernel(8, feature_split=False, has_gather=True)
_agg16_kernel = _seg_kernel(16, feature_split=False, has_gather=True)
_agg32_kernel = _seg_kernel(32, feature_split=True, has_gather=True)


def _fm_feats(xt, fm_w, fm_b, fm_v):
    x1 = xt @ fm_w.T + fm_b
    xv = xt @ fm_v.T
    sum_of_square = (xt * xt) @ (fm_v * fm_v).T
    x2 = 0.5 * jnp.sum(xv * xv - sum_of_square, axis=-1, keepdims=True)
    return x1 + x2


def _bn_relu(z, g, b):
    mu = jnp.mean(z, axis=0)
    var = jnp.var(z, axis=0)
    return jax.nn.relu((z - mu) / jnp.sqrt(var + 1e-5) * g + b)


def kernel(x, edge_index, fm_w, fm_b, fm_v, Wl1, bl1, Wr1, g1, be1,
           Wl2, bl2, Wr2, g2, be2, Wl3, bl3, Wr3, w1, b1, w2, b2):
    src = edge_index[0]
    dst = edge_index[1]

    zrows8 = jnp.zeros((_RPS, 8), jnp.float32)
    zrows16 = jnp.zeros((_RPS, 16), jnp.float32)
    zrows32 = jnp.zeros((_RPS, 32), jnp.float32)
    ones_rows = jnp.ones((_K, 8), jnp.float32)

    degp = _deg_kernel(ones_rows, src, dst, zrows8)
    deg = (degp[0] + degp[1])[:_N, 0]
    invdeg = 1.0 / jnp.maximum(deg, 1.0)
    invdeg_tbl = jnp.broadcast_to(
        jnp.pad(invdeg, (0, _NP - _N))[:, None], (_NP, 8))
    cvp = _cvec_kernel(invdeg_tbl, dst, src, zrows8)
    cvec = (cvp[0] + cvp[1])[:_N, 0]

    xts = x
    xi_all = jnp.concatenate(
        [xts, jax.vmap(lambda xt: _fm_feats(xt, fm_w, fm_b, fm_v))(xts)], axis=2
    )
    cs = 1.0 / jnp.maximum(jnp.max(jnp.abs(xi_all), axis=1), 1e-12)

    xi_tbl = jnp.pad(
        jnp.transpose(xi_all, (1, 0, 2)).reshape(_N, _T * 4),
        ((0, _NP - _N), (0, 0)),
    )

    a1p = _agg16_kernel(xi_tbl, src, dst, zrows16)
    agg1 = (a1p[0] + a1p[1])[:_N]

    scores = []
    for t in range(_T):
        xi_t = xi_all[t]
        a1 = agg1[:, 4 * t:4 * t + 4]
        z1 = (a1 * invdeg[:, None]) @ (Wl1 * cs[t]).T + bl1 \
            + xi_t @ (Wr1 * cs[t]).T
        h1 = _bn_relu(z1, g1, be1)

        h1_tbl = jnp.pad(h1, ((0, _NP - _N), (0, 0)))
        h1_tbl = jnp.transpose(h1_tbl.reshape(_NP, 2, 32), (1, 0, 2))
        a2p = _agg32_kernel(h1_tbl, src, dst, zrows32)
        a2 = jnp.concatenate([a2p[0][:_N], a2p[1][:_N]], axis=1)

        z2 = (a2 * invdeg[:, None]) @ Wl2.T + bl2 + h1 @ Wr2.T
        h2 = _bn_relu(z2, g2, be2)

        sc_sum = (cvec @ h2) @ Wl3[0] + _N * bl3[0] + jnp.sum(h2, axis=0) @ Wr3[0]
        scores.append(sc_sum)

    score = jnp.stack(scores)
    out = jax.nn.relu(score @ w1.T + b1)
    out = out @ w2.T + b2
    return out


# speedup vs baseline: 27.3668x; 3.8111x over previous
import functools

import jax
import jax.numpy as jnp
from jax import lax
from jax.experimental import pallas as pl
from jax.experimental.pallas import tpu as pltpu
from jax.experimental.pallas import tpu_sc as plsc

_N = 50000
_E = 1600000
_T = 4
_H = 64

_NC = 2
_NS = 16
_NP = 50048
_RPS = _NP // _NS
_K = 80

_mesh = plsc.VectorSubcoreMesh(core_axis_name="c", subcore_axis_name="s")


_S = 2000


def _seg_kernel(width, feature_split, has_gather):
    if feature_split:
        epw = _E // _NS
    else:
        epw = _E // (_NC * _NS)
    sup = 800 if width == 32 else _S
    nsub = sup // _K
    n_supers = epw // sup
    assert n_supers * sup == epw

    scratch = [
        pltpu.VMEM((sup,), jnp.int32),
        pltpu.VMEM((nsub, _K), jnp.int32),
        pltpu.VMEM((sup, width), jnp.float32),
        pltpu.VMEM_SHARED((_NP, width), jnp.float32),
        pltpu.SemaphoreType.DMA,
        pltpu.SemaphoreType.DMA,
    ]

    @functools.partial(
        pl.kernel,
        out_type=jax.ShapeDtypeStruct((_NC, _NP, width), jnp.float32),
        mesh=_mesh,
        scratch_types=scratch,
        compiler_params=pltpu.CompilerParams(use_tc_tiling_on_sc=False),
        name=f"sc_seg_w{width}_{'fs' if feature_split else 'es'}",
    )
    def kern(tbl, gidx_hbm, sidx_hbm, zrows, out, gv, sv, rows, acc, gsem, ssem):
        c = lax.axis_index("c")
        s = lax.axis_index("s")
        pltpu.sync_copy(zrows, acc.at[pl.ds(s * _RPS, _RPS)])
        if feature_split:
            tblc = tbl.at[c]
        else:
            tblc = tbl
        if not has_gather:
            pltpu.sync_copy(tblc.at[pl.ds(0, sup)], rows)
        plsc.subcore_barrier()

        if feature_split:
            base0 = s * epw
        else:
            base0 = (s * _NC + c) * epw
        drain_src = tblc.at[pl.ds(0, sup)]

        @pl.loop(0, n_supers)
        def super_chunk(i):
            base = base0 + i * sup
            pltpu.sync_copy(sidx_hbm.at[pl.ds(base // _K, nsub)], sv)
            if has_gather:
                pltpu.sync_copy(gidx_hbm.at[pl.ds(base, sup)], gv)

                @pl.loop(0, nsub)
                def fire_gather(j):
                    sl = pl.ds(j * _K, _K)
                    pltpu.async_copy(tblc.at[gv.at[sl]], rows.at[sl], gsem)

                pltpu.make_async_copy(drain_src, rows, gsem).wait()

            @pl.loop(0, nsub)
            def fire_scatter(j):
                sl = pl.ds(j * _K, _K)
                pltpu.async_copy(rows.at[sl], acc.at[sv.at[j]], ssem, add=True)

            pltpu.make_async_copy(drain_src, rows, ssem).wait()

        plsc.subcore_barrier()
        sl = pl.ds(s * _RPS, _RPS)
        pltpu.sync_copy(acc.at[sl], out.at[c].at[sl])

    return kern


_deg_kernel = _seg_kernel(8, feature_split=False, has_gather=False)
_cvec_kernel = _seg_kernel(8, feature_split=False, has_gather=True)
_agg16_kernel = _seg_kernel(16, feature_split=False, has_gather=True)
_agg32_kernel = _seg_kernel(32, feature_split=True, has_gather=True)


def _fm_feats(xt, fm_w, fm_b, fm_v):
    x1 = xt @ fm_w.T + fm_b
    xv = xt @ fm_v.T
    sum_of_square = (xt * xt) @ (fm_v * fm_v).T
    x2 = 0.5 * jnp.sum(xv * xv - sum_of_square, axis=-1, keepdims=True)
    return x1 + x2


def _bn_relu(z, g, b):
    mu = jnp.mean(z, axis=0)
    var = jnp.var(z, axis=0)
    return jax.nn.relu((z - mu) / jnp.sqrt(var + 1e-5) * g + b)


def kernel(x, edge_index, fm_w, fm_b, fm_v, Wl1, bl1, Wr1, g1, be1,
           Wl2, bl2, Wr2, g2, be2, Wl3, bl3, Wr3, w1, b1, w2, b2):
    src = edge_index[0]
    dst = edge_index[1]
    src2 = src.reshape(_E // _K, _K)
    dst2 = dst.reshape(_E // _K, _K)

    zrows8 = jnp.zeros((_RPS, 8), jnp.float32)
    zrows16 = jnp.zeros((_RPS, 16), jnp.float32)
    zrows32 = jnp.zeros((_RPS, 32), jnp.float32)
    ones_rows = jnp.ones((_S, 8), jnp.float32)

    degp = _deg_kernel(ones_rows, src, dst2, zrows8)
    deg = (degp[0] + degp[1])[:_N, 0]
    invdeg = 1.0 / jnp.maximum(deg, 1.0)
    invdeg_tbl = jnp.broadcast_to(
        jnp.pad(invdeg, (0, _NP - _N))[:, None], (_NP, 8))
    cvp = _cvec_kernel(invdeg_tbl, dst, src2, zrows8)
    cvec = (cvp[0] + cvp[1])[:_N, 0]

    xts = x
    xi_all = jnp.concatenate(
        [xts, jax.vmap(lambda xt: _fm_feats(xt, fm_w, fm_b, fm_v))(xts)], axis=2
    )
    cs = 1.0 / jnp.maximum(jnp.max(jnp.abs(xi_all), axis=1), 1e-12)

    xi_tbl = jnp.pad(
        jnp.transpose(xi_all, (1, 0, 2)).reshape(_N, _T * 4),
        ((0, _NP - _N), (0, 0)),
    )

    a1p = _agg16_kernel(xi_tbl, src, dst2, zrows16)
    agg1 = (a1p[0] + a1p[1])[:_N]

    scores = []
    for t in range(_T):
        xi_t = xi_all[t]
        a1 = agg1[:, 4 * t:4 * t + 4]
        z1 = (a1 * invdeg[:, None]) @ (Wl1 * cs[t]).T + bl1 \
            + xi_t @ (Wr1 * cs[t]).T
        h1 = _bn_relu(z1, g1, be1)

        h1_tbl = jnp.pad(h1, ((0, _NP - _N), (0, 0)))
        h1_tbl = jnp.transpose(h1_tbl.reshape(_NP, 2, 32), (1, 0, 2))
        a2p = _agg32_kernel(h1_tbl, src, dst2, zrows32)
        a2 = jnp.concatenate([a2p[0][:_N], a2p[1][:_N]], axis=1)

        z2 = (a2 * invdeg[:, None]) @ Wl2.T + bl2 + h1 @ Wr2.T
        h2 = _bn_relu(z2, g2, be2)

        sc_sum = (cvec @ h2) @ Wl3[0] + _N * bl3[0] + jnp.sum(h2, axis=0) @ Wr3[0]
        scores.append(sc_sum)

    score = jnp.stack(scores)
    out = jax.nn.relu(score @ w1.T + b1)
    out = out @ w2.T + b2
    return out


# speedup vs baseline: 33.9096x; 1.2391x over previous
import functools

import jax
import jax.numpy as jnp
from jax import lax
from jax.experimental import pallas as pl
from jax.experimental.pallas import tpu as pltpu
from jax.experimental.pallas import tpu_sc as plsc

_N = 50000
_E = 1600000
_T = 4
_H = 64

_NC = 2
_NS = 16
_NP = 50048
_RPS = _NP // _NS
_K = 80

_mesh = plsc.VectorSubcoreMesh(core_axis_name="c", subcore_axis_name="s")


_S = 2000


def _seg_kernel(width, feature_split, has_gather):
    if feature_split:
        epw = _E // _NS
    else:
        epw = _E // (_NC * _NS)
    sup = {32: 800}.get(width, _S)
    nsub = sup // _K
    n_supers = epw // sup
    assert n_supers * sup == epw

    scratch = [
        pltpu.VMEM((sup,), jnp.int32),
        pltpu.VMEM((sup,), jnp.int32),
        pltpu.VMEM((nsub, _K), jnp.int32),
        pltpu.VMEM((nsub, _K), jnp.int32),
        pltpu.VMEM((sup, width), jnp.float32),
        pltpu.VMEM_SHARED((_NP, width), jnp.float32),
        pltpu.SemaphoreType.DMA,
        pltpu.SemaphoreType.DMA,
        pltpu.SemaphoreType.DMA,
    ]

    @functools.partial(
        pl.kernel,
        out_type=jax.ShapeDtypeStruct((_NC, _NP, width), jnp.float32),
        mesh=_mesh,
        scratch_types=scratch,
        compiler_params=pltpu.CompilerParams(use_tc_tiling_on_sc=False),
        name=f"sc_seg_w{width}_{'fs' if feature_split else 'es'}",
    )
    def kern(tbl, gidx_hbm, sidx_hbm, zrows, out, gva, gvb, sva, svb, rows,
             acc, gsem, ssem, isem):
        c = lax.axis_index("c")
        s = lax.axis_index("s")
        pltpu.sync_copy(zrows, acc.at[pl.ds(s * _RPS, _RPS)])
        if feature_split:
            tblc = tbl.at[c]
        else:
            tblc = tbl
        if not has_gather:
            pltpu.sync_copy(tblc.at[pl.ds(0, sup)], rows)
        plsc.subcore_barrier()

        if feature_split:
            base0 = s * epw
        else:
            base0 = (s * _NC + c) * epw
        drain_src = tblc.at[pl.ds(0, sup)]
        sidx_dummy = sidx_hbm.at[pl.ds(0, nsub)]
        gidx_dummy = gidx_hbm.at[pl.ds(0, sup)]

        def idx_fetch(i, gv, sv):
            base = base0 + i * sup
            pltpu.async_copy(sidx_hbm.at[pl.ds(base // _K, nsub)], sv, isem)
            if has_gather:
                pltpu.async_copy(gidx_hbm.at[pl.ds(base, sup)], gv, isem)

        def idx_drain(gv, sv):
            pltpu.make_async_copy(sidx_dummy, sv, isem).wait()
            if has_gather:
                pltpu.make_async_copy(gidx_dummy, gv, isem).wait()

        def process(gv, sv):
            if has_gather:
                @pl.loop(0, nsub)
                def fire_gather(j):
                    sl = pl.ds(j * _K, _K)
                    pltpu.async_copy(tblc.at[gv.at[sl]], rows.at[sl], gsem)

                pltpu.make_async_copy(drain_src, rows, gsem).wait()

            @pl.loop(0, nsub)
            def fire_scatter(j):
                sl = pl.ds(j * _K, _K)
                pltpu.async_copy(rows.at[sl], acc.at[sv.at[j]], ssem, add=True)

            pltpu.make_async_copy(drain_src, rows, ssem).wait()

        n_pairs = n_supers // 2
        odd = n_supers % 2
        idx_fetch(0, gva, sva)

        @pl.loop(0, n_pairs)
        def pair(k):
            i = 2 * k
            idx_drain(gva, sva)
            idx_fetch(i + 1, gvb, svb)
            process(gva, sva)
            idx_drain(gvb, svb)
            nxt = i + 2
            if not odd:
                nxt = jnp.minimum(nxt, n_supers - 1)
            idx_fetch(nxt, gva, sva)
            process(gvb, svb)

        idx_drain(gva, sva)
        if odd:
            process(gva, sva)

        plsc.subcore_barrier()
        sl = pl.ds(s * _RPS, _RPS)
        pltpu.sync_copy(acc.at[sl], out.at[c].at[sl])

    return kern


_deg_kernel = _seg_kernel(8, feature_split=False, has_gather=False)
_cvec_kernel = _seg_kernel(8, feature_split=False, has_gather=True)
_agg16_kernel = _seg_kernel(16, feature_split=False, has_gather=True)
_agg32_kernel = _seg_kernel(32, feature_split=True, has_gather=True)


def _fm_feats(xt, fm_w, fm_b, fm_v):
    x1 = xt @ fm_w.T + fm_b
    xv = xt @ fm_v.T
    sum_of_square = (xt * xt) @ (fm_v * fm_v).T
    x2 = 0.5 * jnp.sum(xv * xv - sum_of_square, axis=-1, keepdims=True)
    return x1 + x2


def _bn_relu(z, g, b):
    mu = jnp.mean(z, axis=0)
    var = jnp.var(z, axis=0)
    return jax.nn.relu((z - mu) / jnp.sqrt(var + 1e-5) * g + b)


def kernel(x, edge_index, fm_w, fm_b, fm_v, Wl1, bl1, Wr1, g1, be1,
           Wl2, bl2, Wr2, g2, be2, Wl3, bl3, Wr3, w1, b1, w2, b2):
    src = edge_index[0]
    dst = edge_index[1]
    src2 = src.reshape(_E // _K, _K)
    dst2 = dst.reshape(_E // _K, _K)

    zrows8 = jnp.zeros((_RPS, 8), jnp.float32)
    zrows16 = jnp.zeros((_RPS, 16), jnp.float32)
    zrows32 = jnp.zeros((_RPS, 32), jnp.float32)
    ones_rows = jnp.ones((_S, 8), jnp.float32)

    xts = x
    xi_all = jnp.concatenate(
        [xts, jax.vmap(lambda xt: _fm_feats(xt, fm_w, fm_b, fm_v))(xts)], axis=2
    )
    cs = 1.0 / jnp.maximum(jnp.max(jnp.abs(xi_all), axis=1), 1e-12)

    xi_tbl = jnp.pad(
        jnp.transpose(xi_all, (1, 0, 2)).reshape(_N, _T * 4),
        ((0, _NP - _N), (0, 0)),
    )

    degp = _deg_kernel(ones_rows, src, dst2, zrows8)
    deg = (degp[0] + degp[1])[:_N, 0]
    invdeg = 1.0 / jnp.maximum(deg, 1.0)

    a1p = _agg16_kernel(xi_tbl, src, dst2, zrows16)
    agg1 = (a1p[0] + a1p[1])[:_N]

    invdeg_tbl = jnp.broadcast_to(
        jnp.pad(invdeg, (0, _NP - _N))[:, None], (_NP, 8))
    cvp = _cvec_kernel(invdeg_tbl, dst, src2, zrows8)
    cvec = (cvp[0] + cvp[1])[:_N, 0]

    scores = []
    for t in range(_T):
        xi_t = xi_all[t]
        a1 = agg1[:, 4 * t:4 * t + 4]
        z1 = (a1 * invdeg[:, None]) @ (Wl1 * cs[t]).T + bl1 \
            + xi_t @ (Wr1 * cs[t]).T
        h1 = _bn_relu(z1, g1, be1)

        h1_tbl = jnp.pad(h1, ((0, _NP - _N), (0, 0)))
        h1_tbl = jnp.transpose(h1_tbl.reshape(_NP, 2, 32), (1, 0, 2))
        a2p = _agg32_kernel(h1_tbl, src, dst2, zrows32)
        a2 = jnp.concatenate([a2p[0][:_N], a2p[1][:_N]], axis=1)

        z2 = (a2 * invdeg[:, None]) @ Wl2.T + bl2 + h1 @ Wr2.T
        h2 = _bn_relu(z2, g2, be2)

        sc_sum = (cvec @ h2) @ Wl3[0] + _N * bl3[0] + jnp.sum(h2, axis=0) @ Wr3[0]
        scores.append(sc_sum)

    score = jnp.stack(scores)
    out = jax.nn.relu(score @ w1.T + b1)
    out = out @ w2.T + b2
    return out


# speedup vs baseline: 34.4949x; 1.0173x over previous
import functools

import jax
import jax.numpy as jnp
from jax import lax
from jax.experimental import pallas as pl
from jax.experimental.pallas import tpu as pltpu
from jax.experimental.pallas import tpu_sc as plsc

_N = 50000
_E = 1600000
_T = 4
_H = 64

_NC = 2
_NS = 16
_NP = 50048
_RPS = _NP // _NS
_K = 80

_mesh = plsc.VectorSubcoreMesh(core_axis_name="c", subcore_axis_name="s")


_S = 2000


def _seg_kernel(width, feature_split, has_gather):
    if feature_split:
        epw = _E // _NS
    else:
        epw = _E // (_NC * _NS)
    sup = {32: 800}.get(width, _S)
    nsub = sup // _K
    n_supers = epw // sup
    assert n_supers * sup == epw

    scratch = [
        pltpu.VMEM((sup,), jnp.int32),
        pltpu.VMEM((sup,), jnp.int32),
        pltpu.VMEM((nsub, _K), jnp.int32),
        pltpu.VMEM((nsub, _K), jnp.int32),
        pltpu.VMEM((sup, width), jnp.float32),
        pltpu.VMEM_SHARED((_NP, width), jnp.float32),
        pltpu.SemaphoreType.DMA,
        pltpu.SemaphoreType.DMA,
        pltpu.SemaphoreType.DMA,
    ]

    @functools.partial(
        pl.kernel,
        out_type=jax.ShapeDtypeStruct((_NC, _NP, width), jnp.float32),
        mesh=_mesh,
        scratch_types=scratch,
        compiler_params=pltpu.CompilerParams(use_tc_tiling_on_sc=False),
        name=f"sc_seg_w{width}_{'fs' if feature_split else 'es'}",
    )
    def kern(tbl, gidx_hbm, sidx_hbm, zrows, out, gva, gvb, sva, svb, rows,
             acc, gsem, ssem, isem):
        c = lax.axis_index("c")
        s = lax.axis_index("s")
        pltpu.sync_copy(zrows, acc.at[pl.ds(s * _RPS, _RPS)])
        if feature_split:
            tblc = tbl.at[c]
        else:
            tblc = tbl
        if not has_gather:
            pltpu.sync_copy(tblc.at[pl.ds(0, sup)], rows)
        plsc.subcore_barrier()

        if feature_split:
            base0 = s * epw
        else:
            base0 = (s * _NC + c) * epw
        drain_src = tblc.at[pl.ds(0, sup)]
        sidx_dummy = sidx_hbm.at[pl.ds(0, nsub)]
        gidx_dummy = gidx_hbm.at[pl.ds(0, sup)]

        def idx_fetch(i, gv, sv):
            base = base0 + i * sup
            pltpu.async_copy(sidx_hbm.at[pl.ds(base // _K, nsub)], sv, isem)
            if has_gather:
                pltpu.async_copy(gidx_hbm.at[pl.ds(base, sup)], gv, isem)

        def idx_drain(gv, sv):
            pltpu.make_async_copy(sidx_dummy, sv, isem).wait()
            if has_gather:
                pltpu.make_async_copy(gidx_dummy, gv, isem).wait()

        def process(gv, sv):
            if has_gather:
                @pl.loop(0, nsub)
                def fire_gather(j):
                    sl = pl.ds(j * _K, _K)
                    pltpu.async_copy(tblc.at[gv.at[sl]], rows.at[sl], gsem)

                pltpu.make_async_copy(drain_src, rows, gsem).wait()

            @pl.loop(0, nsub)
            def fire_scatter(j):
                sl = pl.ds(j * _K, _K)
                pltpu.async_copy(rows.at[sl], acc.at[sv.at[j]], ssem, add=True)

            pltpu.make_async_copy(drain_src, rows, ssem).wait()

        n_pairs = n_supers // 2
        odd = n_supers % 2
        idx_fetch(0, gva, sva)

        @pl.loop(0, n_pairs)
        def pair(k):
            i = 2 * k
            idx_drain(gva, sva)
            idx_fetch(i + 1, gvb, svb)
            process(gva, sva)
            idx_drain(gvb, svb)
            nxt = i + 2
            if not odd:
                nxt = jnp.minimum(nxt, n_supers - 1)
            idx_fetch(nxt, gva, sva)
            process(gvb, svb)

        idx_drain(gva, sva)
        if odd:
            process(gva, sva)

        plsc.subcore_barrier()
        sl = pl.ds(s * _RPS, _RPS)
        pltpu.sync_copy(acc.at[sl], out.at[c].at[sl])

    return kern


_deg_kernel = _seg_kernel(8, feature_split=False, has_gather=False)
_cvec_kernel = _seg_kernel(8, feature_split=False, has_gather=True)
_agg16_kernel = _seg_kernel(16, feature_split=False, has_gather=True)
_agg32_kernel = _seg_kernel(32, feature_split=True, has_gather=True)


_BN_BLK = 2000
_GRID = _N // _BN_BLK


def _row_spec(width):
    return pl.BlockSpec((_BN_BLK, width), lambda i: (i, 0))


def _acc_spec(width):
    return pl.BlockSpec((1, width), lambda i: (0, 0))


def _full(shape):
    return pl.BlockSpec(shape, lambda i: tuple(0 for _ in shape))


def _xi_kernel(x12, A, B, C, D, bvec):
    def body(x_r, a_r, b_r, c_r, d_r, bv_r, xi_r, mx_r):
        xb = x_r[...]
        xv = jnp.dot(xb, b_r[...], preferred_element_type=jnp.float32)
        sos = jnp.dot(xb * xb, c_r[...], preferred_element_type=jnp.float32)
        xi = (jnp.dot(xb, a_r[...], preferred_element_type=jnp.float32)
              + bv_r[...]
              + jnp.dot(xv * xv - sos, d_r[...],
                        preferred_element_type=jnp.float32))
        xi_r[...] = xi

        @pl.when(pl.program_id(0) == 0)
        def _():
            mx_r[...] = jnp.zeros_like(mx_r)

        mx_r[...] = jnp.maximum(mx_r[...], jnp.max(jnp.abs(xi), axis=0,
                                                   keepdims=True))

    return pl.pallas_call(
        body,
        grid=(_GRID,),
        in_specs=[_row_spec(12), _full((12, 16)), _full((12, 12)),
                  _full((12, 12)), _full((12, 16)), _full((1, 16))],
        out_specs=[pl.BlockSpec((_BN_BLK, 16), lambda i: (i, 0)),
                   _acc_spec(16)],
        out_shape=[jax.ShapeDtypeStruct((_NP, 16), jnp.float32),
                   jax.ShapeDtypeStruct((1, 16), jnp.float32)],
    )(x12, A, B, C, D, bvec)


def _z1_stats_kernel(agg1, xi_tbl, invdeg8, wl_t, wr_t, bl):
    def body(a_r, x_r, d_r, wl_r, wr_r, bl_r, s_r):
        inv = d_r[:, 0:4]

        @pl.when(pl.program_id(0) == 0)
        def _():
            s_r[...] = jnp.zeros_like(s_r)

        for t in range(_T):
            a1 = a_r[:, 4 * t:4 * t + 4] * inv
            xi = x_r[:, 4 * t:4 * t + 4]
            z = (jnp.dot(a1, wl_r[t], preferred_element_type=jnp.float32)
                 + jnp.dot(xi, wr_r[t], preferred_element_type=jnp.float32)
                 + bl_r[...])
            s_r[2 * t, :] += jnp.sum(z, axis=0)
            s_r[2 * t + 1, :] += jnp.sum(z * z, axis=0)

    return pl.pallas_call(
        body,
        grid=(_GRID,),
        in_specs=[_row_spec(16), _row_spec(16), _row_spec(8),
                  _full((_T, 4, _H)), _full((_T, 4, _H)), _full((1, _H))],
        out_specs=pl.BlockSpec((2 * _T, _H), lambda i: (0, 0)),
        out_shape=jax.ShapeDtypeStruct((2 * _T, _H), jnp.float32),
    )(agg1, xi_tbl, invdeg8, wl_t, wr_t, bl)


def _h1_kernel(t, agg1, xi_tbl, invdeg8, wl_t, wr_t, bl, mu, rstd, g, be):
    def body(a_r, x_r, d_r, wl_r, wr_r, bl_r, mu_r, rs_r, g_r, be_r, h_r):
        inv = d_r[:, 0:4]
        a1 = a_r[:, 4 * t:4 * t + 4] * inv
        xi = x_r[:, 4 * t:4 * t + 4]
        z = (jnp.dot(a1, wl_r[t], preferred_element_type=jnp.float32)
             + jnp.dot(xi, wr_r[t], preferred_element_type=jnp.float32)
             + bl_r[...])
        h = jax.nn.relu((z - mu_r[...]) * rs_r[...] * g_r[...] + be_r[...])
        h_r[0] = h[:, :32]
        h_r[1] = h[:, 32:]

    return pl.pallas_call(
        body,
        grid=(_GRID,),
        in_specs=[_row_spec(16), _row_spec(16), _row_spec(8),
                  _full((_T, 4, _H)), _full((_T, 4, _H)), _full((1, _H)),
                  _full((1, _H)), _full((1, _H)), _full((1, _H)),
                  _full((1, _H))],
        out_specs=pl.BlockSpec((2, _BN_BLK, 32), lambda i: (0, i, 0)),
        out_shape=jax.ShapeDtypeStruct((2, _NP, 32), jnp.float32),
    )(agg1, xi_tbl, invdeg8, wl_t, wr_t, bl, mu, rstd, g, be)


def _z2_stats_kernel(a2_tbl, h1_tbl, invdeg64, wl_t, wr_t, bl):
    def body(a_r, h_r, d_r, wl_r, wr_r, bl_r, s_r):
        a2 = jnp.concatenate([a_r[0], a_r[1]], axis=1) * d_r[...]
        h1 = jnp.concatenate([h_r[0], h_r[1]], axis=1)
        z = (jnp.dot(a2, wl_r[...], preferred_element_type=jnp.float32)
             + jnp.dot(h1, wr_r[...], preferred_element_type=jnp.float32)
             + bl_r[...])

        @pl.when(pl.program_id(0) == 0)
        def _():
            s_r[...] = jnp.zeros_like(s_r)

        s_r[0, :] += jnp.sum(z, axis=0)
        s_r[1, :] += jnp.sum(z * z, axis=0)

    tbl_spec = pl.BlockSpec((2, _BN_BLK, 32), lambda i: (0, i, 0))
    return pl.pallas_call(
        body,
        grid=(_GRID,),
        in_specs=[tbl_spec, tbl_spec, _row_spec(_H),
                  _full((_H, _H)), _full((_H, _H)), _full((1, _H))],
        out_specs=pl.BlockSpec((2, _H), lambda i: (0, 0)),
        out_shape=jax.ShapeDtypeStruct((2, _H), jnp.float32),
    )(a2_tbl, h1_tbl, invdeg64, wl_t, wr_t, bl)


def _score_kernel(a2_tbl, h1_tbl, invdeg64, cvec64, wl_t, wr_t, bl, mu, rstd,
                  g, be):
    def body(a_r, h_r, d_r, c_r, wl_r, wr_r, bl_r, mu_r, rs_r, g_r, be_r,
             s_r):
        a2 = jnp.concatenate([a_r[0], a_r[1]], axis=1) * d_r[...]
        h1 = jnp.concatenate([h_r[0], h_r[1]], axis=1)
        z = (jnp.dot(a2, wl_r[...], preferred_element_type=jnp.float32)
             + jnp.dot(h1, wr_r[...], preferred_element_type=jnp.float32)
             + bl_r[...])
        h2 = jax.nn.relu((z - mu_r[...]) * rs_r[...] * g_r[...] + be_r[...])

        @pl.when(pl.program_id(0) == 0)
        def _():
            s_r[...] = jnp.zeros_like(s_r)

        s_r[0, :] += jnp.sum(c_r[...] * h2, axis=0)
        s_r[1, :] += jnp.sum(h2, axis=0)

    tbl_spec = pl.BlockSpec((2, _BN_BLK, 32), lambda i: (0, i, 0))
    return pl.pallas_call(
        body,
        grid=(_GRID,),
        in_specs=[tbl_spec, tbl_spec, _row_spec(_H), _row_spec(_H),
                  _full((_H, _H)), _full((_H, _H)), _full((1, _H)),
                  _full((1, _H)), _full((1, _H)), _full((1, _H)),
                  _full((1, _H))],
        out_specs=pl.BlockSpec((2, _H), lambda i: (0, 0)),
        out_shape=jax.ShapeDtypeStruct((2, _H), jnp.float32),
    )(a2_tbl, h1_tbl, invdeg64, cvec64, wl_t, wr_t, bl, mu, rstd, g, be)


def kernel(x, edge_index, fm_w, fm_b, fm_v, Wl1, bl1, Wr1, g1, be1,
           Wl2, bl2, Wr2, g2, be2, Wl3, bl3, Wr3, w1, b1, w2, b2):
    src = edge_index[0]
    dst = edge_index[1]
    src2 = src.reshape(_E // _K, _K)
    dst2 = dst.reshape(_E // _K, _K)

    zrows8 = jnp.zeros((_RPS, 8), jnp.float32)
    zrows16 = jnp.zeros((_RPS, 16), jnp.float32)
    zrows32 = jnp.zeros((_RPS, 32), jnp.float32)
    ones_rows = jnp.ones((_S, 8), jnp.float32)

    x12 = jnp.transpose(x, (1, 0, 2)).reshape(_N, _T * 3)
    A = jnp.zeros((12, 16), jnp.float32)
    B = jnp.zeros((12, 12), jnp.float32)
    C = jnp.zeros((12, 12), jnp.float32)
    D = jnp.zeros((12, 16), jnp.float32)
    bvec = jnp.zeros((1, 16), jnp.float32)
    for t in range(_T):
        A = A.at[3 * t:3 * t + 3, 4 * t:4 * t + 3].set(jnp.eye(3))
        A = A.at[3 * t:3 * t + 3, 4 * t + 3].set(fm_w[0])
        B = B.at[3 * t:3 * t + 3, 3 * t:3 * t + 3].set(fm_v.T)
        C = C.at[3 * t:3 * t + 3, 3 * t:3 * t + 3].set((fm_v * fm_v).T)
        D = D.at[3 * t:3 * t + 3, 4 * t + 3].set(0.5)
        bvec = bvec.at[0, 4 * t + 3].set(fm_b[0])
    xi_tbl, colmax = _xi_kernel(x12, A, B, C, D, bvec)

    cs = (1.0 / jnp.maximum(colmax, 1e-12)).reshape(_T, 4)
    wl1_t = jnp.transpose(Wl1[None, :, :] * cs[:, None, :], (0, 2, 1))
    wr1_t = jnp.transpose(Wr1[None, :, :] * cs[:, None, :], (0, 2, 1))

    degp = _deg_kernel(ones_rows, src, dst2, zrows8)
    invdeg8 = 1.0 / jnp.maximum(degp[0] + degp[1], 1.0)

    cvp = _cvec_kernel(invdeg8, dst, src2, zrows8)
    cvec64 = jnp.broadcast_to((cvp[0] + cvp[1])[:, 0:1], (_NP, _H))
    invdeg64 = jnp.broadcast_to(invdeg8[:, 0:1], (_NP, _H))

    a1p = _agg16_kernel(xi_tbl, src, dst2, zrows16)
    agg1 = a1p[0] + a1p[1]

    bl1r = bl1.reshape(1, _H)
    s12 = _z1_stats_kernel(agg1, xi_tbl, invdeg8, wl1_t, wr1_t, bl1r)
    mu1 = s12[0::2] / _N
    var1 = s12[1::2] / _N - mu1 * mu1
    rstd1 = jax.lax.rsqrt(var1 + 1e-5)

    g1r, be1r = g1.reshape(1, _H), be1.reshape(1, _H)
    bl2r, g2r, be2r = bl2.reshape(1, _H), g2.reshape(1, _H), be2.reshape(1, _H)
    wl2_t, wr2_t = Wl2.T, Wr2.T

    scores = []
    for t in range(_T):
        h1_tbl = _h1_kernel(t, agg1, xi_tbl, invdeg8, wl1_t, wr1_t, bl1r,
                            mu1[t:t + 1], rstd1[t:t + 1], g1r, be1r)
        a2p = _agg32_kernel(h1_tbl, src, dst2, zrows32)
        s2 = _z2_stats_kernel(a2p, h1_tbl, invdeg64, wl2_t, wr2_t, bl2r)
        mu2 = s2[0:1] / _N
        var2 = s2[1:2] / _N - mu2 * mu2
        rstd2 = jax.lax.rsqrt(var2 + 1e-5)
        pq = _score_kernel(a2p, h1_tbl, invdeg64, cvec64, wl2_t, wr2_t, bl2r,
                           mu2, rstd2, g2r, be2r)
        sc_sum = pq[0] @ Wl3[0] + _N * bl3[0] + pq[1] @ Wr3[0]
        scores.append(sc_sum)

    score = jnp.stack(scores)
    out = jax.nn.relu(score @ w1.T + b1)
    out = out @ w2.T + b2
    return out
